# baseline (speedup 1.0000x reference)
import functools

import jax
import jax.numpy as jnp
from jax import lax
from jax.experimental import pallas as pl
from jax.experimental.pallas import tpu as pltpu
from jax.experimental.pallas import tpu_sc as plsc

_B = 16384
_D = 128
_NC = 2
_NS = 16
_NW = _NC * _NS
_BW = _B // _NW
_C = 128
_NCHUNK = _BW // _C
_G = _C // 16


def _body(users_hbm, items_hbm, utab_hbm, itab_hbm, out_hbm,
          uidx, iidx, ubuf, ibuf, tbuf, outv, sem0, sem1, sem2, sem3):
    wid = lax.axis_index("s") * _NC + lax.axis_index("c")
    base = pl.multiple_of(wid * _BW, _BW)

    lane = lax.iota(jnp.int32, 16)

    sems = (sem0, sem1, sem3)

    chunks = ((0, 32), (32, 96), (128, 128), (256, 128), (384, 128))

    def mk(i, s):
        off, n = chunks[i]
        cu = pltpu.make_async_copy(
            utab_hbm.at[uidx.at[pl.ds(off, n)]],
            ubuf.at[s, pl.ds(0, n)], sems[s])
        ci = pltpu.make_async_copy(
            itab_hbm.at[iidx.at[pl.ds(off, n)]],
            ibuf.at[s, pl.ds(0, n)], sems[s])
        return cu, ci

    def start(i, s):
        cu, ci = mk(i, s)
        cu.start()
        ci.start()

    col_base = lane * 16

    def compute(i, s):
        coff, n = chunks[i]
        ub = ubuf.at[s]
        ib = ibuf.at[s]

        def dot_row(row):
            a = ub[row, pl.ds(0, 16)] * ib[row, pl.ds(0, 16)]
            b = ub[row, pl.ds(16, 16)] * ib[row, pl.ds(16, 16)]
            for k in range(2, 8, 2):
                a = a + ub[row, pl.ds(16 * k, 16)] * ib[row, pl.ds(16 * k, 16)]
                b = b + ub[row, pl.ds(16 * (k + 1), 16)] * ib[row, pl.ds(16 * (k + 1), 16)]
            return a + b

        def group(g, carry):
            for p in range(0, 16, 4):
                acc0 = dot_row(g * 16 + p)
                acc1 = dot_row(g * 16 + p + 1)
                acc2 = dot_row(g * 16 + p + 2)
                acc3 = dot_row(g * 16 + p + 3)
                tbuf[pl.ds(16 * p, 16)] = acc0
                tbuf[pl.ds(16 * (p + 1), 16)] = acc1
                tbuf[pl.ds(16 * (p + 2), 16)] = acc2
                tbuf[pl.ds(16 * (p + 3), 16)] = acc3
            d0 = plsc.load_gather(tbuf, [col_base])
            d1 = plsc.load_gather(tbuf, [col_base + 1])
            d2 = plsc.load_gather(tbuf, [col_base + 2])
            d3 = plsc.load_gather(tbuf, [col_base + 3])
            for l in range(4, 16, 4):
                d0 = d0 + plsc.load_gather(tbuf, [col_base + l])
                d1 = d1 + plsc.load_gather(tbuf, [col_base + l + 1])
                d2 = d2 + plsc.load_gather(tbuf, [col_base + l + 2])
                d3 = d3 + plsc.load_gather(tbuf, [col_base + l + 3])
            dots = (d0 + d1) + (d2 + d3)
            res = 1.0 / (1.0 + jnp.exp(-dots))
            off = pl.multiple_of(coff + g * 16, 16)
            outv[pl.ds(off, 16)] = res
            return carry

        lax.fori_loop(0, n // 16, group, 0)

    n0 = chunks[0][1]
    iu0 = pltpu.make_async_copy(
        users_hbm.at[pl.ds(base, n0)], uidx.at[pl.ds(0, n0)], sem2)
    ii0 = pltpu.make_async_copy(
        items_hbm.at[pl.ds(base, n0)], iidx.at[pl.ds(0, n0)], sem2)
    iu0.start()
    ii0.start()
    iu0.wait()
    ii0.wait()
    start(0, 0)

    rbase = pl.multiple_of(base + n0, n0)
    iur = pltpu.make_async_copy(
        users_hbm.at[pl.ds(rbase, _BW - n0)], uidx.at[pl.ds(n0, _BW - n0)], sem2)
    iir = pltpu.make_async_copy(
        items_hbm.at[pl.ds(rbase, _BW - n0)], iidx.at[pl.ds(n0, _BW - n0)], sem2)
    iur.start()
    iir.start()
    iur.wait()
    iir.wait()
    start(1, 1)
    start(2, 2)

    owbs = []
    for i in range(len(chunks)):
        s = i % 3
        cu, ci = mk(i, s)
        cu.wait()
        ci.wait()
        compute(i, s)
        if i + 3 < len(chunks):
            start(i + 3, s)
        coff, n = chunks[i]
        ow = pltpu.make_async_copy(
            outv.at[pl.ds(coff, n)], out_hbm.at[pl.ds(base + coff, n)], sem2)
        ow.start()
        owbs.append(ow)
    for ow in owbs:
        ow.wait()


@functools.partial(
    pl.kernel,
    out_type=jax.ShapeDtypeStruct((_B,), jnp.float32),
    mesh=plsc.VectorSubcoreMesh(
        core_axis_name="c", subcore_axis_name="s",
        num_cores=_NC, num_subcores=_NS),
    compiler_params=pltpu.CompilerParams(
        needs_layout_passes=False,
        skip_device_barrier=True,
        disable_bounds_checks=True,
        disable_semaphore_checks=True,
    ),
    scratch_types=[
        pltpu.VMEM((_BW,), jnp.int32),
        pltpu.VMEM((_BW,), jnp.int32),
        pltpu.VMEM((3, _C, _D), jnp.float32),
        pltpu.VMEM((3, _C, _D), jnp.float32),
        pltpu.VMEM((256,), jnp.float32),
        pltpu.VMEM((_BW,), jnp.float32),
        pltpu.SemaphoreType.DMA,
        pltpu.SemaphoreType.DMA,
        pltpu.SemaphoreType.DMA,
        pltpu.SemaphoreType.DMA,
    ],
)
def _ultragcn_sc(users_hbm, items_hbm, utab_hbm, itab_hbm, out_hbm,
                 uidx, iidx, ubuf, ibuf, tbuf, outv, sem0, sem1, sem2, sem3):
    _body(users_hbm, items_hbm, utab_hbm, itab_hbm, out_hbm,
          uidx, iidx, ubuf, ibuf, tbuf, outv, sem0, sem1, sem2, sem3)


def kernel(data, user_table, item_table):
    return _ultragcn_sc(data[:, 0], data[:, 1], user_table, item_table)


# speedup vs baseline: 1.0518x; 1.0518x over previous
import functools

import jax
import jax.numpy as jnp
from jax import lax
from jax.experimental import pallas as pl
from jax.experimental.pallas import tpu as pltpu
from jax.experimental.pallas import tpu_sc as plsc

_B = 16384
_D = 128
_NC = 2
_NS = 16
_NW = _NC * _NS
_BW = _B // _NW
_C = 128
_NCHUNK = _BW // _C
_G = _C // 16


def _body(users_hbm, items_hbm, utab_hbm, itab_hbm, out_hbm,
          uidx, iidx, ubuf, ibuf, tbuf, outv, sem0, sem1, sem2):
    wid = lax.axis_index("s") * _NC + lax.axis_index("c")
    base = pl.multiple_of(wid * _BW, _BW)

    lane = lax.iota(jnp.int32, 16)

    sems = (sem0, sem1)

    chunks = ((0, 32), (32, 96), (128, 128), (256, 128), (384, 128))

    def mk(i, s):
        off, n = chunks[i]
        cu = pltpu.make_async_copy(
            utab_hbm.at[uidx.at[pl.ds(off, n)]],
            ubuf.at[s, pl.ds(0, n)], sems[s])
        ci = pltpu.make_async_copy(
            itab_hbm.at[iidx.at[pl.ds(off, n)]],
            ibuf.at[s, pl.ds(0, n)], sems[s])
        return cu, ci

    def start(i, s):
        cu, ci = mk(i, s)
        cu.start()
        ci.start()

    col_base = lane * 16

    def compute(i, s):
        coff, n = chunks[i]
        ub = ubuf.at[s]
        ib = ibuf.at[s]

        def dot_row(row):
            a = ub[row, pl.ds(0, 16)] * ib[row, pl.ds(0, 16)]
            b = ub[row, pl.ds(16, 16)] * ib[row, pl.ds(16, 16)]
            for k in range(2, 8, 2):
                a = a + ub[row, pl.ds(16 * k, 16)] * ib[row, pl.ds(16 * k, 16)]
                b = b + ub[row, pl.ds(16 * (k + 1), 16)] * ib[row, pl.ds(16 * (k + 1), 16)]
            return a + b

        def group(g, carry):
            for p in range(0, 16, 4):
                acc0 = dot_row(g * 16 + p)
                acc1 = dot_row(g * 16 + p + 1)
                acc2 = dot_row(g * 16 + p + 2)
                acc3 = dot_row(g * 16 + p + 3)
                tbuf[pl.ds(16 * p, 16)] = acc0
                tbuf[pl.ds(16 * (p + 1), 16)] = acc1
                tbuf[pl.ds(16 * (p + 2), 16)] = acc2
                tbuf[pl.ds(16 * (p + 3), 16)] = acc3
            d0 = plsc.load_gather(tbuf, [col_base])
            d1 = plsc.load_gather(tbuf, [col_base + 1])
            d2 = plsc.load_gather(tbuf, [col_base + 2])
            d3 = plsc.load_gather(tbuf, [col_base + 3])
            for l in range(4, 16, 4):
                d0 = d0 + plsc.load_gather(tbuf, [col_base + l])
                d1 = d1 + plsc.load_gather(tbuf, [col_base + l + 1])
                d2 = d2 + plsc.load_gather(tbuf, [col_base + l + 2])
                d3 = d3 + plsc.load_gather(tbuf, [col_base + l + 3])
            dots = (d0 + d1) + (d2 + d3)
            res = 1.0 / (1.0 + jnp.exp(-dots))
            off = pl.multiple_of(coff + g * 16, 16)
            outv[pl.ds(off, 16)] = res
            return carry

        lax.fori_loop(0, n // 16, group, 0)

    n0 = chunks[0][1]
    iu0 = pltpu.make_async_copy(
        users_hbm.at[pl.ds(base, n0)], uidx.at[pl.ds(0, n0)], sem2)
    ii0 = pltpu.make_async_copy(
        items_hbm.at[pl.ds(base, n0)], iidx.at[pl.ds(0, n0)], sem2)
    iu0.start()
    ii0.start()
    iu0.wait()
    ii0.wait()
    start(0, 0)

    rbase = pl.multiple_of(base + n0, n0)
    iur = pltpu.make_async_copy(
        users_hbm.at[pl.ds(rbase, _BW - n0)], uidx.at[pl.ds(n0, _BW - n0)], sem2)
    iir = pltpu.make_async_copy(
        items_hbm.at[pl.ds(rbase, _BW - n0)], iidx.at[pl.ds(n0, _BW - n0)], sem2)
    iur.start()
    iir.start()
    iur.wait()
    iir.wait()
    start(1, 1)

    owbs = []
    for i in range(len(chunks)):
        s = i % 2
        cu, ci = mk(i, s)
        cu.wait()
        ci.wait()
        compute(i, s)
        if i + 2 < len(chunks):
            start(i + 2, s)
        coff, n = chunks[i]
        ow = pltpu.make_async_copy(
            outv.at[pl.ds(coff, n)], out_hbm.at[pl.ds(base + coff, n)], sem2)
        ow.start()
        owbs.append(ow)
    for ow in owbs:
        ow.wait()


@functools.partial(
    pl.kernel,
    out_type=jax.ShapeDtypeStruct((_B,), jnp.float32),
    mesh=plsc.VectorSubcoreMesh(
        core_axis_name="c", subcore_axis_name="s",
        num_cores=_NC, num_subcores=_NS),
    compiler_params=pltpu.CompilerParams(
        needs_layout_passes=False,
        skip_device_barrier=True,
        disable_bounds_checks=True,
        disable_semaphore_checks=True,
    ),
    scratch_types=[
        pltpu.VMEM((_BW,), jnp.int32),
        pltpu.VMEM((_BW,), jnp.int32),
        pltpu.VMEM((2, _C, _D), jnp.float32),
        pltpu.VMEM((2, _C, _D), jnp.float32),
        pltpu.VMEM((256,), jnp.float32),
        pltpu.VMEM((_BW,), jnp.float32),
        pltpu.SemaphoreType.DMA,
        pltpu.SemaphoreType.DMA,
        pltpu.SemaphoreType.DMA,
    ],
)
def _ultragcn_sc(users_hbm, items_hbm, utab_hbm, itab_hbm, out_hbm,
                 uidx, iidx, ubuf, ibuf, tbuf, outv, sem0, sem1, sem2):
    _body(users_hbm, items_hbm, utab_hbm, itab_hbm, out_hbm,
          uidx, iidx, ubuf, ibuf, tbuf, outv, sem0, sem1, sem2)


def kernel(data, user_table, item_table):
    return _ultragcn_sc(data[:, 0], data[:, 1], user_table, item_table)
